# baseline (speedup 1.0000x reference)
import jax
import jax.numpy as jnp
from jax import lax
from jax.experimental import pallas as pl
from jax.experimental.pallas import tpu as pltpu
from jax.experimental.pallas import tpu_sc as plsc

DIM = 32
BATCH = 16384
NUM_CORES = 2
NUM_SUBCORES = 16
NUM_WORKERS = NUM_CORES * NUM_SUBCORES
ROWS_PER_WORKER = BATCH // NUM_WORKERS
CHUNK = 128
NUM_CHUNKS = ROWS_PER_WORKER // CHUNK
GROUPS = ROWS_PER_WORKER // 16


def _bpr_body(user_idx_hbm, pos_idx_hbm, neg_idx_hbm, user_emb_hbm,
              item_emb_hbm, pos_out_hbm, neg_out_hbm,
              idx_u, idx_i, idx_j, rows_u, rows_i, rows_j,
              out_p, out_n, sem):
    wid = lax.axis_index("s") * NUM_CORES + lax.axis_index("c")
    base = wid * ROWS_PER_WORKER

    for c in range(NUM_CHUNKS):
        off = base + c * CHUNK
        pltpu.sync_copy(user_idx_hbm.at[pl.ds(off, CHUNK)], idx_u.at[c])
        pltpu.sync_copy(pos_idx_hbm.at[pl.ds(off, CHUNK)], idx_i.at[c])
        pltpu.sync_copy(neg_idx_hbm.at[pl.ds(off, CHUNK)], idx_j.at[c])

    copies = []
    for c in range(NUM_CHUNKS):
        dst = pl.ds(c * CHUNK, CHUNK)
        copies.append(pltpu.async_copy(
            user_emb_hbm.at[idx_u.at[c]], rows_u.at[dst], sem))
        copies.append(pltpu.async_copy(
            item_emb_hbm.at[idx_i.at[c]], rows_i.at[dst], sem))
        copies.append(pltpu.async_copy(
            item_emb_hbm.at[idx_j.at[c]], rows_j.at[dst], sem))
    for cp in copies:
        cp.wait()

    lane = lax.iota(jnp.int32, 16)

    def group_body(g, _):
        row = g * 16 + lane
        accp = jnp.zeros((16,), jnp.float32)
        accn = jnp.zeros((16,), jnp.float32)
        for d in range(DIM):
            col = jnp.full((16,), d, jnp.int32)
            gu = plsc.load_gather(rows_u, [row, col])
            gi = plsc.load_gather(rows_i, [row, col])
            gj = plsc.load_gather(rows_j, [row, col])
            accp = accp + gu * gi
            accn = accn + gu * gj
        out_p[pl.ds(g * 16, 16)] = accp
        out_n[pl.ds(g * 16, 16)] = accn
        return 0

    lax.fori_loop(0, GROUPS, group_body, 0)

    pltpu.sync_copy(out_p, pos_out_hbm.at[pl.ds(base, ROWS_PER_WORKER)])
    pltpu.sync_copy(out_n, neg_out_hbm.at[pl.ds(base, ROWS_PER_WORKER)])


@jax.jit
def _bpr_sc(batch_user, batch_pos_item, batch_neg_item, user_emb, item_emb):
    mesh = plsc.VectorSubcoreMesh(core_axis_name="c", subcore_axis_name="s")
    kfn = pl.kernel(
        _bpr_body,
        out_type=(
            jax.ShapeDtypeStruct((BATCH,), jnp.float32),
            jax.ShapeDtypeStruct((BATCH,), jnp.float32),
        ),
        mesh=mesh,
        compiler_params=pltpu.CompilerParams(
            needs_layout_passes=False, use_tc_tiling_on_sc=False),
        scratch_types=[
            pltpu.VMEM((NUM_CHUNKS, CHUNK), jnp.int32),
            pltpu.VMEM((NUM_CHUNKS, CHUNK), jnp.int32),
            pltpu.VMEM((NUM_CHUNKS, CHUNK), jnp.int32),
            pltpu.VMEM((ROWS_PER_WORKER, DIM), jnp.float32),
            pltpu.VMEM((ROWS_PER_WORKER, DIM), jnp.float32),
            pltpu.VMEM((ROWS_PER_WORKER, DIM), jnp.float32),
            pltpu.VMEM((ROWS_PER_WORKER,), jnp.float32),
            pltpu.VMEM((ROWS_PER_WORKER,), jnp.float32),
            pltpu.SemaphoreType.DMA,
        ],
    )
    pos, neg = kfn(batch_user, batch_pos_item, batch_neg_item,
                   user_emb, item_emb)
    return pos.reshape(BATCH, 1), neg.reshape(BATCH, 1)


def kernel(batch_user, batch_pos_item, batch_neg_item, user_emb, item_emb):
    return _bpr_sc(batch_user, batch_pos_item, batch_neg_item,
                   user_emb, item_emb)
